# baseline (speedup 1.0000x reference)
import functools

import jax
import jax.numpy as jnp
from jax import lax
from jax.experimental import pallas as pl
from jax.experimental.pallas import tpu as pltpu
from jax.experimental.pallas import tpu_sc as plsc

B = 4096
F = 26
V = 1000
D = 128
DNN_IN = F * D
H1, H2 = 1024, 512
PK = D
XW = F * 2 * D

NC = 2
NS = 16
NW = NC * NS
BPW = B // NW
IPW = BPW * F
CH = 4
CHI = CH * F
NCHUNK = BPW // CH

BB = 256
NBLK = B // BB


def _sc_gather(packed_tab, lin_tab, idx, lin_idx):
    mesh = plsc.VectorSubcoreMesh(core_axis_name="c", subcore_axis_name="s",
                                  num_cores=NC, num_subcores=NS)

    @functools.partial(
        pl.kernel,
        mesh=mesh,
        compiler_params=pltpu.CompilerParams(needs_layout_passes=False),
        out_type=(
            jax.ShapeDtypeStruct((B * F, PK), jnp.int32),
            jax.ShapeDtypeStruct((B,), jnp.float32),
        ),
        scratch_types=[
            pltpu.VMEM((IPW,), jnp.int32),
            pltpu.VMEM((CHI, PK), jnp.int32),
            pltpu.VMEM((CHI, PK), jnp.int32),
            pltpu.VMEM((F * V,), jnp.float32),
            pltpu.VMEM((F, BPW), jnp.int32),
            pltpu.VMEM((BPW,), jnp.float32),
            pltpu.SemaphoreType.DMA,
            pltpu.SemaphoreType.DMA,
        ],
    )
    def k(tab_hbm, lin_hbm, idx_hbm, lin_idx_hbm,
          rows_out, lin_out,
          idx_v, buf0, buf1, lin_tab_v, lin_idx_v, lin_sum_v, sem0, sem1):
        wid = lax.axis_index("s") * NC + lax.axis_index("c")
        base = wid * IPW

        pltpu.sync_copy(idx_hbm.at[pl.ds(base, IPW)], idx_v)
        pltpu.sync_copy(lin_hbm, lin_tab_v)
        pltpu.sync_copy(lin_idx_hbm.at[:, pl.ds(wid * BPW, BPW)], lin_idx_v)

        for g in range(BPW // 16):
            acc = jnp.zeros((16,), jnp.float32)
            for f in range(F):
                iv = lin_idx_v[f, pl.ds(g * 16, 16)]
                acc = acc + plsc.load_gather(lin_tab_v, [iv])
            lin_sum_v[pl.ds(g * 16, 16)] = acc
        pltpu.sync_copy(lin_sum_v, lin_out.at[pl.ds(wid * BPW, BPW)])

        pltpu.async_copy(tab_hbm.at[idx_v.at[pl.ds(0, CHI)]], buf0, sem0)

        def pair(p, _):
            j0 = 2 * p
            pltpu.make_async_copy(
                tab_hbm.at[idx_v.at[pl.ds(j0 * CHI, CHI)]], buf0, sem0).wait()
            pltpu.async_copy(
                tab_hbm.at[idx_v.at[pl.ds((j0 + 1) * CHI, CHI)]], buf1, sem1)
            pltpu.sync_copy(buf0, rows_out.at[pl.ds(base + j0 * CHI, CHI)])
            pltpu.make_async_copy(
                tab_hbm.at[idx_v.at[pl.ds((j0 + 1) * CHI, CHI)]], buf1,
                sem1).wait()
            nxt = jnp.minimum(j0 + 2, NCHUNK - 1) * CHI
            pltpu.async_copy(tab_hbm.at[idx_v.at[pl.ds(nxt, CHI)]], buf0,
                             sem0)
            pltpu.sync_copy(buf1,
                            rows_out.at[pl.ds(base + (j0 + 1) * CHI, CHI)])
            return 0

        lax.fori_loop(0, NCHUNK // 2, pair, 0)
        pltpu.make_async_copy(
            tab_hbm.at[idx_v.at[pl.ds((NCHUNK - 1) * CHI, CHI)]], buf0,
            sem0).wait()

    return k(packed_tab, lin_tab, idx, lin_idx)


def _tc_body(x_ref, lin_ref, w1_ref, b1_ref, w2_ref, b2_ref,
             w3_ref, scal_ref, out_ref):
    h = jnp.zeros((BB, H1), jnp.float32)
    s = jnp.zeros((BB, D), jnp.float32)
    q = jnp.zeros((BB, D), jnp.float32)
    for f in range(F):
        xd = x_ref[:, f * 2 * D:f * 2 * D + D]
        h = h + jnp.dot(xd, w1_ref[f * D:(f + 1) * D, :],
                        preferred_element_type=jnp.float32)
        e = x_ref[:, f * 2 * D + D:(f + 1) * 2 * D].astype(jnp.float32)
        s = s + e
        q = q + e * e
    h = jnp.maximum(h + b1_ref[...], 0.0)
    h = jnp.dot(h.astype(jnp.bfloat16), w2_ref[...],
                preferred_element_type=jnp.float32)
    h = jnp.maximum(h + b2_ref[...], 0.0)
    dnn_mat = jnp.dot(h.astype(jnp.bfloat16), w3_ref[...],
                      preferred_element_type=jnp.float32)
    dnn = jnp.sum(dnn_mat, axis=1)

    fm_term = 0.5 * jnp.sum(s * s - q, axis=1)

    bias0 = scal_ref[0]
    b3 = scal_ref[1]
    wf0 = scal_ref[2]
    wf1 = scal_ref[3]
    bf = scal_ref[4]
    fm_output = bias0 + lin_ref[...] + fm_term
    logit = wf0 * fm_output + wf1 * (dnn + b3) + bf
    out_ref[...] = 1.0 / (1.0 + jnp.exp(-logit))


def _tc_forward(x, lin_sum, W1, b1, W2, b2, W3p, scal):
    return pl.pallas_call(
        _tc_body,
        grid=(NBLK,),
        in_specs=[
            pl.BlockSpec((BB, XW), lambda i: (i, 0)),
            pl.BlockSpec((BB,), lambda i: (i,)),
            pl.BlockSpec((DNN_IN, H1), lambda i: (0, 0)),
            pl.BlockSpec((1, H1), lambda i: (0, 0)),
            pl.BlockSpec((H1, H2), lambda i: (0, 0)),
            pl.BlockSpec((1, H2), lambda i: (0, 0)),
            pl.BlockSpec((H2, D), lambda i: (0, 0)),
            pl.BlockSpec(memory_space=pltpu.SMEM),
        ],
        out_specs=pl.BlockSpec((BB,), lambda i: (i,)),
        out_shape=jax.ShapeDtypeStruct((B,), jnp.float32),
    )(x, lin_sum, W1, b1, W2, b2, W3p, scal)


def kernel(features, emb_fm, lin_fm, bias, emb_dnn, W1, b1, W2, b2, W3, b3,
           Wf, bf):
    feats = features.astype(jnp.int32)
    offs = (jnp.arange(F, dtype=jnp.int32) * V)[None, :]
    idx = (feats + offs).reshape(B * F)
    lin_idx = feats.T + (jnp.arange(F, dtype=jnp.int32) * V)[:, None]

    dnn_i32 = lax.bitcast_convert_type(
        emb_dnn.astype(jnp.bfloat16).reshape(F * V, D // 2, 2), jnp.int32)
    fm_i32 = lax.bitcast_convert_type(
        emb_fm.astype(jnp.bfloat16).reshape(F * V, D // 2, 2), jnp.int32)
    packed = jnp.concatenate([dnn_i32, fm_i32], axis=1)

    rows_i32, lin_sum = _sc_gather(packed, lin_fm.reshape(F * V), idx, lin_idx)

    x = lax.bitcast_convert_type(rows_i32, jnp.bfloat16).reshape(B, XW)

    W3p = jnp.pad(W3, ((0, 0), (0, D - 1)))
    scal = jnp.concatenate([bias, b3, Wf[0], Wf[1], bf])
    out = _tc_forward(x, lin_sum, W1.astype(jnp.bfloat16),
                      b1.reshape(1, H1), W2.astype(jnp.bfloat16),
                      b2.reshape(1, H2), W3p.astype(jnp.bfloat16), scal)
    return out


# speedup vs baseline: 58.3914x; 58.3914x over previous
import functools

import jax
import jax.numpy as jnp
from jax import lax
from jax.experimental import pallas as pl
from jax.experimental.pallas import tpu as pltpu
from jax.experimental.pallas import tpu_sc as plsc

B = 4096
F = 26
V = 1000
D = 128
DNN_IN = F * D
H1, H2 = 1024, 512

NC = 2
NS = 16
NW = NC * NS
BPW = B // NW
IPW = BPW * F
CH = 4
CHI = CH * F
NCHUNK = BPW // CH

BB = 256
NBLK = B // BB


def _sc_gather(dnn_tab, fm_tab, lin_tab, idx, lin_idx):
    mesh = plsc.VectorSubcoreMesh(core_axis_name="c", subcore_axis_name="s",
                                  num_cores=NC, num_subcores=NS)

    @functools.partial(
        pl.kernel,
        mesh=mesh,
        compiler_params=pltpu.CompilerParams(needs_layout_passes=False),
        out_type=(
            jax.ShapeDtypeStruct((B * F, D), jnp.float32),
            jax.ShapeDtypeStruct((B * F, D), jnp.float32),
            jax.ShapeDtypeStruct((B,), jnp.float32),
        ),
        scratch_types=[
            pltpu.VMEM((IPW,), jnp.int32),
            pltpu.VMEM((CHI, D), jnp.float32),
            pltpu.VMEM((CHI, D), jnp.float32),
            pltpu.VMEM((CHI, D), jnp.float32),
            pltpu.VMEM((CHI, D), jnp.float32),
            pltpu.VMEM((F * V,), jnp.float32),
            pltpu.VMEM((F, BPW), jnp.int32),
            pltpu.VMEM((BPW,), jnp.float32),
            pltpu.SemaphoreType.DMA,
            pltpu.SemaphoreType.DMA,
            pltpu.SemaphoreType.DMA,
            pltpu.SemaphoreType.DMA,
        ],
    )
    def k(dnn_hbm, fm_hbm, lin_hbm, idx_hbm, lin_idx_hbm,
          dnn_out, fm_out, lin_out,
          idx_v, d0, f0, d1, f1, lin_tab_v, lin_idx_v, lin_sum_v,
          sd0, sf0, sd1, sf1):
        wid = lax.axis_index("s") * NC + lax.axis_index("c")
        base = wid * IPW

        pltpu.sync_copy(idx_hbm.at[pl.ds(base, IPW)], idx_v)
        pltpu.sync_copy(lin_hbm, lin_tab_v)
        pltpu.sync_copy(lin_idx_hbm.at[:, pl.ds(wid * BPW, BPW)], lin_idx_v)

        def gather(j, dbuf, fbuf, dsem, fsem):
            pltpu.async_copy(dnn_hbm.at[idx_v.at[pl.ds(j * CHI, CHI)]],
                             dbuf, dsem)
            pltpu.async_copy(fm_hbm.at[idx_v.at[pl.ds(j * CHI, CHI)]],
                             fbuf, fsem)

        def wait(j, dbuf, fbuf, dsem, fsem):
            pltpu.make_async_copy(dnn_hbm.at[idx_v.at[pl.ds(j * CHI, CHI)]],
                                  dbuf, dsem).wait()
            pltpu.make_async_copy(fm_hbm.at[idx_v.at[pl.ds(j * CHI, CHI)]],
                                  fbuf, fsem).wait()

        def writeback(j, dbuf, fbuf):
            pltpu.sync_copy(dbuf, dnn_out.at[pl.ds(base + j * CHI, CHI)])
            pltpu.sync_copy(fbuf, fm_out.at[pl.ds(base + j * CHI, CHI)])

        gather(0, d0, f0, sd0, sf0)
        for g in range(BPW // 16):
            acc = jnp.zeros((16,), jnp.float32)
            for f in range(F):
                iv = lin_idx_v[f, pl.ds(g * 16, 16)]
                acc = acc + plsc.load_gather(lin_tab_v, [iv])
            lin_sum_v[pl.ds(g * 16, 16)] = acc
        pltpu.sync_copy(lin_sum_v, lin_out.at[pl.ds(wid * BPW, BPW)])

        def pair(p, _):
            j0 = 2 * p
            wait(j0, d0, f0, sd0, sf0)
            gather(j0 + 1, d1, f1, sd1, sf1)
            writeback(j0, d0, f0)
            wait(j0 + 1, d1, f1, sd1, sf1)
            nxt = jnp.minimum(j0 + 2, NCHUNK - 1)
            gather(nxt, d0, f0, sd0, sf0)
            writeback(j0 + 1, d1, f1)
            return 0

        lax.fori_loop(0, NCHUNK // 2, pair, 0)
        wait(NCHUNK - 1, d0, f0, sd0, sf0)

    return k(dnn_tab, fm_tab, lin_tab, idx, lin_idx)


def _tc_body(dnn_x_ref, fm_x_ref, lin_ref, w1_ref, b1_ref, w2_ref, b2_ref,
             w3_ref, scal_ref, out_ref):
    x = dnn_x_ref[...].astype(jnp.bfloat16)
    h = jnp.dot(x, w1_ref[...], preferred_element_type=jnp.float32)
    h = jnp.maximum(h + b1_ref[...], 0.0)
    h = jnp.dot(h.astype(jnp.bfloat16), w2_ref[...],
                preferred_element_type=jnp.float32)
    h = jnp.maximum(h + b2_ref[...], 0.0)
    dnn_mat = jnp.dot(h.astype(jnp.bfloat16), w3_ref[...],
                      preferred_element_type=jnp.float32)
    dnn = jnp.sum(dnn_mat, axis=1)

    fm_x = fm_x_ref[...]
    s = jnp.zeros((BB, D), jnp.float32)
    q = jnp.zeros((BB, D), jnp.float32)
    for f in range(F):
        e = fm_x[:, f * D:(f + 1) * D]
        s = s + e
        q = q + e * e
    fm_term = 0.5 * jnp.sum(s * s - q, axis=1)

    bias0 = scal_ref[0]
    b3 = scal_ref[1]
    wf0 = scal_ref[2]
    wf1 = scal_ref[3]
    bf = scal_ref[4]
    fm_output = bias0 + lin_ref[...] + fm_term
    logit = wf0 * fm_output + wf1 * (dnn + b3) + bf
    out_ref[...] = 1.0 / (1.0 + jnp.exp(-logit))


def _tc_forward(dnn_x, fm_x, lin_sum, W1, b1, W2, b2, W3p, scal):
    return pl.pallas_call(
        _tc_body,
        grid=(NBLK,),
        in_specs=[
            pl.BlockSpec((BB, DNN_IN), lambda i: (i, 0)),
            pl.BlockSpec((BB, DNN_IN), lambda i: (i, 0)),
            pl.BlockSpec((BB,), lambda i: (i,)),
            pl.BlockSpec((DNN_IN, H1), lambda i: (0, 0)),
            pl.BlockSpec((1, H1), lambda i: (0, 0)),
            pl.BlockSpec((H1, H2), lambda i: (0, 0)),
            pl.BlockSpec((1, H2), lambda i: (0, 0)),
            pl.BlockSpec((H2, D), lambda i: (0, 0)),
            pl.BlockSpec(memory_space=pltpu.SMEM),
        ],
        out_specs=pl.BlockSpec((BB,), lambda i: (i,)),
        out_shape=jax.ShapeDtypeStruct((B,), jnp.float32),
    )(dnn_x, fm_x, lin_sum, W1, b1, W2, b2, W3p, scal)


def kernel(features, emb_fm, lin_fm, bias, emb_dnn, W1, b1, W2, b2, W3, b3,
           Wf, bf):
    feats = features.astype(jnp.int32)
    offs = (jnp.arange(F, dtype=jnp.int32) * V)[None, :]
    idx = (feats + offs).reshape(B * F)
    lin_idx = feats.T + (jnp.arange(F, dtype=jnp.int32) * V)[:, None]

    d_rows, f_rows, lin_sum = _sc_gather(
        emb_dnn.reshape(F * V, D), emb_fm.reshape(F * V, D),
        lin_fm.reshape(F * V), idx, lin_idx)

    dnn_x = d_rows.reshape(B, DNN_IN)
    fm_x = f_rows.reshape(B, DNN_IN)

    W3p = jnp.pad(W3, ((0, 0), (0, D - 1)))
    scal = jnp.concatenate([bias, b3, Wf[0], Wf[1], bf])
    out = _tc_forward(dnn_x, fm_x, lin_sum, W1.astype(jnp.bfloat16),
                      b1.reshape(1, H1), W2.astype(jnp.bfloat16),
                      b2.reshape(1, H2), W3p.astype(jnp.bfloat16), scal)
    return out


# speedup vs baseline: 85.7062x; 1.4678x over previous
import functools

import jax
import jax.numpy as jnp
from jax import lax
from jax.experimental import pallas as pl
from jax.experimental.pallas import tpu as pltpu
from jax.experimental.pallas import tpu_sc as plsc

B = 4096
F = 26
V = 1000
D = 128
DNN_IN = F * D
H1, H2 = 1024, 512

NC = 2
NS = 16
NW = NC * NS
BPW = B // NW
IPW = BPW * F
CH = 4
CHI = CH * F
NCHUNK = BPW // CH

BB = 256
NBLK = B // BB


def _sc_gather(dnn_tab, fm_tab, lin_tab, idx, lin_idx):
    mesh = plsc.VectorSubcoreMesh(core_axis_name="c", subcore_axis_name="s",
                                  num_cores=NC, num_subcores=NS)

    @functools.partial(
        pl.kernel,
        mesh=mesh,
        compiler_params=pltpu.CompilerParams(needs_layout_passes=False),
        out_type=(
            jax.ShapeDtypeStruct((B * F, D), jnp.float32),
            jax.ShapeDtypeStruct((B * F, D), jnp.float32),
            jax.ShapeDtypeStruct((B,), jnp.float32),
        ),
        scratch_types=[
            pltpu.VMEM((IPW,), jnp.int32),
            pltpu.VMEM((CHI, D), jnp.float32),
            pltpu.VMEM((CHI, D), jnp.float32),
            pltpu.VMEM((CHI, D), jnp.float32),
            pltpu.VMEM((CHI, D), jnp.float32),
            pltpu.VMEM((F * V,), jnp.float32),
            pltpu.VMEM((F, BPW), jnp.int32),
            pltpu.VMEM((BPW,), jnp.float32),
            pltpu.SemaphoreType.DMA,
            pltpu.SemaphoreType.DMA,
            pltpu.SemaphoreType.DMA,
            pltpu.SemaphoreType.DMA,
        ],
    )
    def k(dnn_hbm, fm_hbm, lin_hbm, idx_hbm, lin_idx_hbm,
          dnn_out, fm_out, lin_out,
          idx_v, d0, f0, d1, f1, lin_tab_v, lin_idx_v, lin_sum_v,
          sd0, sf0, sd1, sf1):
        wid = lax.axis_index("s") * NC + lax.axis_index("c")
        base = wid * IPW

        pltpu.sync_copy(idx_hbm.at[pl.ds(base, IPW)], idx_v)
        pltpu.sync_copy(lin_hbm, lin_tab_v)
        pltpu.sync_copy(lin_idx_hbm.at[:, pl.ds(wid * BPW, BPW)], lin_idx_v)

        def gather(j, dbuf, fbuf, dsem, fsem):
            pltpu.async_copy(dnn_hbm.at[idx_v.at[pl.ds(j * CHI, CHI)]],
                             dbuf, dsem)
            pltpu.async_copy(fm_hbm.at[idx_v.at[pl.ds(j * CHI, CHI)]],
                             fbuf, fsem)

        def wait(j, dbuf, fbuf, dsem, fsem):
            pltpu.make_async_copy(dnn_hbm.at[idx_v.at[pl.ds(j * CHI, CHI)]],
                                  dbuf, dsem).wait()
            pltpu.make_async_copy(fm_hbm.at[idx_v.at[pl.ds(j * CHI, CHI)]],
                                  fbuf, fsem).wait()

        def writeback(j, dbuf, fbuf):
            pltpu.sync_copy(dbuf, dnn_out.at[pl.ds(base + j * CHI, CHI)])
            pltpu.sync_copy(fbuf, fm_out.at[pl.ds(base + j * CHI, CHI)])

        gather(0, d0, f0, sd0, sf0)
        for g in range(BPW // 16):
            acc = jnp.zeros((16,), jnp.float32)
            for f in range(F):
                iv = lin_idx_v[f, pl.ds(g * 16, 16)]
                acc = acc + plsc.load_gather(lin_tab_v, [iv])
            lin_sum_v[pl.ds(g * 16, 16)] = acc
        pltpu.sync_copy(lin_sum_v, lin_out.at[pl.ds(wid * BPW, BPW)])

        def pair(p, _):
            j0 = 2 * p
            wait(j0, d0, f0, sd0, sf0)
            gather(j0 + 1, d1, f1, sd1, sf1)
            writeback(j0, d0, f0)
            wait(j0 + 1, d1, f1, sd1, sf1)
            nxt = jnp.minimum(j0 + 2, NCHUNK - 1)
            gather(nxt, d0, f0, sd0, sf0)
            writeback(j0 + 1, d1, f1)
            return 0

        lax.fori_loop(0, NCHUNK // 2, pair, 0)
        wait(NCHUNK - 1, d0, f0, sd0, sf0)

    return k(dnn_tab, fm_tab, lin_tab, idx, lin_idx)


def _tc_body(dnn_x_ref, fm_x_ref, lin_ref, w1_ref, b1_ref, w2_ref, b2_ref,
             w3_ref, scal_ref, out_ref):
    h = jnp.zeros((BB, H1), jnp.float32)
    s = jnp.zeros((BB, D), jnp.float32)
    q = jnp.zeros((BB, D), jnp.float32)
    for f in range(F):
        xd = dnn_x_ref[f].astype(jnp.bfloat16)
        h = h + jnp.dot(xd, w1_ref[f * D:(f + 1) * D, :],
                        preferred_element_type=jnp.float32)
        e = fm_x_ref[f]
        s = s + e
        q = q + e * e
    h = jnp.maximum(h + b1_ref[...], 0.0)
    h = jnp.dot(h.astype(jnp.bfloat16), w2_ref[...],
                preferred_element_type=jnp.float32)
    h = jnp.maximum(h + b2_ref[...], 0.0)
    dnn_mat = jnp.dot(h.astype(jnp.bfloat16), w3_ref[...],
                      preferred_element_type=jnp.float32)
    dnn = jnp.sum(dnn_mat, axis=1)

    fm_term = 0.5 * jnp.sum(s * s - q, axis=1)

    bias0 = scal_ref[0]
    b3 = scal_ref[1]
    wf0 = scal_ref[2]
    wf1 = scal_ref[3]
    bf = scal_ref[4]
    fm_output = bias0 + lin_ref[...] + fm_term
    logit = wf0 * fm_output + wf1 * (dnn + b3) + bf
    out_ref[...] = 1.0 / (1.0 + jnp.exp(-logit))


def _tc_forward(dnn_x, fm_x, lin_sum, W1, b1, W2, b2, W3p, scal):
    return pl.pallas_call(
        _tc_body,
        grid=(NBLK,),
        in_specs=[
            pl.BlockSpec((F, BB, D), lambda i: (0, i, 0)),
            pl.BlockSpec((F, BB, D), lambda i: (0, i, 0)),
            pl.BlockSpec((BB,), lambda i: (i,)),
            pl.BlockSpec((DNN_IN, H1), lambda i: (0, 0)),
            pl.BlockSpec((1, H1), lambda i: (0, 0)),
            pl.BlockSpec((H1, H2), lambda i: (0, 0)),
            pl.BlockSpec((1, H2), lambda i: (0, 0)),
            pl.BlockSpec((H2, D), lambda i: (0, 0)),
            pl.BlockSpec(memory_space=pltpu.SMEM),
        ],
        out_specs=pl.BlockSpec((BB,), lambda i: (i,)),
        out_shape=jax.ShapeDtypeStruct((B,), jnp.float32),
    )(dnn_x, fm_x, lin_sum, W1, b1, W2, b2, W3p, scal)


def kernel(features, emb_fm, lin_fm, bias, emb_dnn, W1, b1, W2, b2, W3, b3,
           Wf, bf):
    feats = features.astype(jnp.int32)
    lin_idx = feats.T + (jnp.arange(F, dtype=jnp.int32) * V)[:, None]
    idx = lin_idx.reshape(F * B)

    d_rows, f_rows, lin_sum = _sc_gather(
        emb_dnn.reshape(F * V, D), emb_fm.reshape(F * V, D),
        lin_fm.reshape(F * V), idx, lin_idx)

    dnn_x = d_rows.reshape(F, B, D)
    fm_x = f_rows.reshape(F, B, D)

    W3p = jnp.pad(W3, ((0, 0), (0, D - 1)))
    scal = jnp.concatenate([bias, b3, Wf[0], Wf[1], bf])
    out = _tc_forward(dnn_x, fm_x, lin_sum, W1.astype(jnp.bfloat16),
                      b1.reshape(1, H1), W2.astype(jnp.bfloat16),
                      b2.reshape(1, H2), W3p.astype(jnp.bfloat16), scal)
    return out


# speedup vs baseline: 99.4218x; 1.1600x over previous
import functools

import jax
import jax.numpy as jnp
from jax import lax
from jax.experimental import pallas as pl
from jax.experimental.pallas import tpu as pltpu
from jax.experimental.pallas import tpu_sc as plsc

B = 4096
F = 26
V = 1000
D = 128
DNN_IN = F * D
H1, H2 = 1024, 512

NC = 2
NS = 16
NW = NC * NS
BPW = B // NW
IPW = BPW * F
CH = 4
CHI = CH * F
NCHUNK = BPW // CH

BB = 256
NBLK = B // BB


def _sc_gather(dnn_tab, fm_tab, lin_tab, idx, lin_idx):
    mesh = plsc.VectorSubcoreMesh(core_axis_name="c", subcore_axis_name="s",
                                  num_cores=NC, num_subcores=NS)

    @functools.partial(
        pl.kernel,
        mesh=mesh,
        compiler_params=pltpu.CompilerParams(needs_layout_passes=False),
        out_type=(
            jax.ShapeDtypeStruct((B * F, D), jnp.float32),
            jax.ShapeDtypeStruct((B * F, D), jnp.float32),
            jax.ShapeDtypeStruct((B,), jnp.float32),
        ),
        scratch_types=[
            pltpu.VMEM((IPW,), jnp.int32),
            pltpu.VMEM((CHI, D), jnp.float32),
            pltpu.VMEM((CHI, D), jnp.float32),
            pltpu.VMEM((CHI, D), jnp.float32),
            pltpu.VMEM((CHI, D), jnp.float32),
            pltpu.VMEM((F * V,), jnp.float32),
            pltpu.VMEM((F, BPW), jnp.int32),
            pltpu.VMEM((BPW,), jnp.float32),
            pltpu.SemaphoreType.DMA,
            pltpu.SemaphoreType.DMA,
            pltpu.SemaphoreType.DMA,
            pltpu.SemaphoreType.DMA,
        ],
    )
    def k(dnn_hbm, fm_hbm, lin_hbm, idx_hbm, lin_idx_hbm,
          dnn_out, fm_out, lin_out,
          idx_v, d0, f0, d1, f1, lin_tab_v, lin_idx_v, lin_sum_v,
          sd0, sf0, sd1, sf1):
        wid = lax.axis_index("s") * NC + lax.axis_index("c")
        base = wid * IPW

        pltpu.sync_copy(idx_hbm.at[pl.ds(base, IPW)], idx_v)
        pltpu.sync_copy(lin_hbm, lin_tab_v)
        pltpu.sync_copy(lin_idx_hbm.at[:, pl.ds(wid * BPW, BPW)], lin_idx_v)

        def gather(j, dbuf, fbuf, dsem, fsem):
            pltpu.async_copy(dnn_hbm.at[idx_v.at[pl.ds(j * CHI, CHI)]],
                             dbuf, dsem)
            pltpu.async_copy(fm_hbm.at[idx_v.at[pl.ds(j * CHI, CHI)]],
                             fbuf, fsem)

        def wait(j, dbuf, fbuf, dsem, fsem):
            pltpu.make_async_copy(dnn_hbm.at[idx_v.at[pl.ds(j * CHI, CHI)]],
                                  dbuf, dsem).wait()
            pltpu.make_async_copy(fm_hbm.at[idx_v.at[pl.ds(j * CHI, CHI)]],
                                  fbuf, fsem).wait()

        def writeback(j, dbuf, fbuf):
            pltpu.sync_copy(dbuf, dnn_out.at[pl.ds(base + j * CHI, CHI)])
            pltpu.sync_copy(fbuf, fm_out.at[pl.ds(base + j * CHI, CHI)])

        gather(0, d0, f0, sd0, sf0)
        for g in range(BPW // 16):
            acc = jnp.zeros((16,), jnp.float32)
            for f in range(F):
                iv = lin_idx_v[f, pl.ds(g * 16, 16)]
                acc = acc + plsc.load_gather(lin_tab_v, [iv])
            lin_sum_v[pl.ds(g * 16, 16)] = acc
        pltpu.sync_copy(lin_sum_v, lin_out.at[pl.ds(wid * BPW, BPW)])

        def pair(p, _):
            j0 = 2 * p
            wait(j0, d0, f0, sd0, sf0)
            gather(j0 + 1, d1, f1, sd1, sf1)
            writeback(j0, d0, f0)
            wait(j0 + 1, d1, f1, sd1, sf1)
            nxt = jnp.minimum(j0 + 2, NCHUNK - 1)
            gather(nxt, d0, f0, sd0, sf0)
            writeback(j0 + 1, d1, f1)
            return 0

        lax.fori_loop(0, NCHUNK // 2, pair, 0)
        wait(NCHUNK - 1, d0, f0, sd0, sf0)

    return k(dnn_tab, fm_tab, lin_tab, idx, lin_idx)


def _tc_body(dnn_x_ref, fm_x_ref, lin_ref, w1_ref, b1_ref, w2_ref, b2_ref,
             w3_ref, scal_ref, out_ref):
    s = jnp.zeros((BB, D), jnp.float32)
    q = jnp.zeros((BB, D), jnp.float32)
    for f in range(F):
        e = fm_x_ref[f]
        s = s + e
        q = q + e * e
    xd = jnp.concatenate(
        [dnn_x_ref[f].astype(jnp.bfloat16) for f in range(F)], axis=1)
    h = jnp.dot(xd, w1_ref[...], preferred_element_type=jnp.float32)
    h = jnp.maximum(h + b1_ref[...], 0.0)
    h = jnp.dot(h.astype(jnp.bfloat16), w2_ref[...],
                preferred_element_type=jnp.float32)
    h = jnp.maximum(h + b2_ref[...], 0.0)
    dnn_mat = jnp.dot(h.astype(jnp.bfloat16), w3_ref[...],
                      preferred_element_type=jnp.float32)
    dnn = jnp.sum(dnn_mat, axis=1)

    fm_term = 0.5 * jnp.sum(s * s - q, axis=1)

    bias0 = scal_ref[0]
    b3 = scal_ref[1]
    wf0 = scal_ref[2]
    wf1 = scal_ref[3]
    bf = scal_ref[4]
    fm_output = bias0 + lin_ref[...] + fm_term
    logit = wf0 * fm_output + wf1 * (dnn + b3) + bf
    out_ref[...] = 1.0 / (1.0 + jnp.exp(-logit))


def _tc_forward(dnn_x, fm_x, lin_sum, W1, b1, W2, b2, W3p, scal):
    return pl.pallas_call(
        _tc_body,
        grid=(NBLK,),
        in_specs=[
            pl.BlockSpec((F, BB, D), lambda i: (0, i, 0)),
            pl.BlockSpec((F, BB, D), lambda i: (0, i, 0)),
            pl.BlockSpec((BB,), lambda i: (i,)),
            pl.BlockSpec((DNN_IN, H1), lambda i: (0, 0)),
            pl.BlockSpec((1, H1), lambda i: (0, 0)),
            pl.BlockSpec((H1, H2), lambda i: (0, 0)),
            pl.BlockSpec((1, H2), lambda i: (0, 0)),
            pl.BlockSpec((H2, D), lambda i: (0, 0)),
            pl.BlockSpec(memory_space=pltpu.SMEM),
        ],
        out_specs=pl.BlockSpec((BB,), lambda i: (i,)),
        out_shape=jax.ShapeDtypeStruct((B,), jnp.float32),
    )(dnn_x, fm_x, lin_sum, W1, b1, W2, b2, W3p, scal)


def kernel(features, emb_fm, lin_fm, bias, emb_dnn, W1, b1, W2, b2, W3, b3,
           Wf, bf):
    feats = features.astype(jnp.int32)
    lin_idx = feats.T + (jnp.arange(F, dtype=jnp.int32) * V)[:, None]
    idx = lin_idx.reshape(F * B)

    d_rows, f_rows, lin_sum = _sc_gather(
        emb_dnn.reshape(F * V, D), emb_fm.reshape(F * V, D),
        lin_fm.reshape(F * V), idx, lin_idx)

    dnn_x = d_rows.reshape(F, B, D)
    fm_x = f_rows.reshape(F, B, D)

    W3p = jnp.pad(W3, ((0, 0), (0, D - 1)))
    scal = jnp.concatenate([bias, b3, Wf[0], Wf[1], bf])
    out = _tc_forward(dnn_x, fm_x, lin_sum, W1.astype(jnp.bfloat16),
                      b1.reshape(1, H1), W2.astype(jnp.bfloat16),
                      b2.reshape(1, H2), W3p.astype(jnp.bfloat16), scal)
    return out


# speedup vs baseline: 102.2808x; 1.0288x over previous
import functools

import jax
import jax.numpy as jnp
from jax import lax
from jax.experimental import pallas as pl
from jax.experimental.pallas import tpu as pltpu
from jax.experimental.pallas import tpu_sc as plsc

B = 4096
F = 26
V = 1000
D = 128
DNN_IN = F * D
H1, H2 = 1024, 512
HD = D // 2

NC = 2
NS = 16
NW = NC * NS
BPW = B // NW
IPW = BPW * F
CH = 4
CHI = CH * F
NCHUNK = BPW // CH

RB = 2600
BB = 256
NBLK = B // BB

def _pack_body(dnn_ref, fm_ref, out_ref):
    def words(x):
        xb = x.astype(jnp.bfloat16).astype(jnp.float32)
        u = lax.bitcast_convert_type(xb, jnp.uint32)
        return (u[:, :HD] >> 16) | u[:, HD:]
    w = jnp.concatenate([words(dnn_ref[...]), words(fm_ref[...])], axis=1)
    out_ref[...] = lax.bitcast_convert_type(w, jnp.int32)


def _pack(dnn_tab, fm_tab):
    return pl.pallas_call(
        _pack_body,
        grid=(F * V // RB,),
        in_specs=[
            pl.BlockSpec((RB, D), lambda i: (i, 0)),
            pl.BlockSpec((RB, D), lambda i: (i, 0)),
        ],
        out_specs=pl.BlockSpec((RB, D), lambda i: (i, 0)),
        out_shape=jax.ShapeDtypeStruct((F * V, D), jnp.int32),
    )(dnn_tab, fm_tab)


def _sc_gather(packed_tab, lin_tab, idx, lin_idx):
    mesh = plsc.VectorSubcoreMesh(core_axis_name="c", subcore_axis_name="s",
                                  num_cores=NC, num_subcores=NS)

    @functools.partial(
        pl.kernel,
        mesh=mesh,
        compiler_params=pltpu.CompilerParams(needs_layout_passes=False),
        out_type=(
            jax.ShapeDtypeStruct((F * B, D), jnp.int32),
            jax.ShapeDtypeStruct((B,), jnp.float32),
        ),
        scratch_types=[
            pltpu.VMEM((IPW,), jnp.int32),
            pltpu.VMEM((CHI, D), jnp.int32),
            pltpu.VMEM((CHI, D), jnp.int32),
            pltpu.VMEM((F * V,), jnp.float32),
            pltpu.VMEM((F, BPW), jnp.int32),
            pltpu.VMEM((BPW,), jnp.float32),
            pltpu.SemaphoreType.DMA,
            pltpu.SemaphoreType.DMA,
        ],
    )
    def k(tab_hbm, lin_hbm, idx_hbm, lin_idx_hbm,
          rows_out, lin_out,
          idx_v, b0, b1, lin_tab_v, lin_idx_v, lin_sum_v, s0, s1):
        wid = lax.axis_index("s") * NC + lax.axis_index("c")
        base = wid * IPW

        pltpu.sync_copy(idx_hbm.at[pl.ds(base, IPW)], idx_v)
        pltpu.sync_copy(lin_hbm, lin_tab_v)
        pltpu.sync_copy(lin_idx_hbm.at[:, pl.ds(wid * BPW, BPW)], lin_idx_v)

        def gather(j, buf, sem):
            pltpu.async_copy(tab_hbm.at[idx_v.at[pl.ds(j * CHI, CHI)]],
                             buf, sem)

        def wait(j, buf, sem):
            pltpu.make_async_copy(tab_hbm.at[idx_v.at[pl.ds(j * CHI, CHI)]],
                                  buf, sem).wait()

        def writeback(j, buf):
            pltpu.sync_copy(buf, rows_out.at[pl.ds(base + j * CHI, CHI)])

        gather(0, b0, s0)
        for g in range(BPW // 16):
            acc = jnp.zeros((16,), jnp.float32)
            for f in range(F):
                iv = lin_idx_v[f, pl.ds(g * 16, 16)]
                acc = acc + plsc.load_gather(lin_tab_v, [iv])
            lin_sum_v[pl.ds(g * 16, 16)] = acc
        pltpu.sync_copy(lin_sum_v, lin_out.at[pl.ds(wid * BPW, BPW)])

        def pair(p, _):
            j0 = 2 * p
            wait(j0, b0, s0)
            gather(j0 + 1, b1, s1)
            writeback(j0, b0)
            wait(j0 + 1, b1, s1)
            gather(jnp.minimum(j0 + 2, NCHUNK - 1), b0, s0)
            writeback(j0 + 1, b1)
            return 0

        lax.fori_loop(0, NCHUNK // 2, pair, 0)
        wait(NCHUNK - 1, b0, s0)

    return k(packed_tab, lin_tab, idx, lin_idx)


def _tc_body(x_ref, lin_ref, w1_ref, b1_ref, w2_ref, b2_ref,
             w3_ref, scal_ref, out_ref):
    def unpack(w):
        lo = lax.bitcast_convert_type(w << 16, jnp.float32)
        hi = lax.bitcast_convert_type(w & jnp.uint32(0xFFFF0000),
                                      jnp.float32)
        return jnp.concatenate([lo, hi], axis=1)

    s = jnp.zeros((BB, D), jnp.float32)
    q = jnp.zeros((BB, D), jnp.float32)
    dnn_parts = []
    for f in range(F):
        w = lax.bitcast_convert_type(x_ref[f], jnp.uint32)
        dnn_parts.append(unpack(w[:, :HD]).astype(jnp.bfloat16))
        e = unpack(w[:, HD:])
        s = s + e
        q = q + e * e
    fm_term = 0.5 * jnp.sum(s * s - q, axis=1)

    xd = jnp.concatenate(dnn_parts, axis=1)
    h = jnp.dot(xd, w1_ref[...], preferred_element_type=jnp.float32)
    h = jnp.maximum(h + b1_ref[...], 0.0)
    h = jnp.dot(h.astype(jnp.bfloat16), w2_ref[...],
                preferred_element_type=jnp.float32)
    h = jnp.maximum(h + b2_ref[...], 0.0)
    dnn_mat = jnp.dot(h.astype(jnp.bfloat16), w3_ref[...],
                      preferred_element_type=jnp.float32)
    dnn = jnp.sum(dnn_mat, axis=1)

    bias0 = scal_ref[0]
    b3 = scal_ref[1]
    wf0 = scal_ref[2]
    wf1 = scal_ref[3]
    bf = scal_ref[4]
    fm_output = bias0 + lin_ref[...] + fm_term
    logit = wf0 * fm_output + wf1 * (dnn + b3) + bf
    out_ref[...] = 1.0 / (1.0 + jnp.exp(-logit))


def _tc_forward(x, lin_sum, W1, b1, W2, b2, W3p, scal):
    return pl.pallas_call(
        _tc_body,
        grid=(NBLK,),
        in_specs=[
            pl.BlockSpec((F, BB, D), lambda i: (0, i, 0)),
            pl.BlockSpec((BB,), lambda i: (i,)),
            pl.BlockSpec((DNN_IN, H1), lambda i: (0, 0)),
            pl.BlockSpec((1, H1), lambda i: (0, 0)),
            pl.BlockSpec((H1, H2), lambda i: (0, 0)),
            pl.BlockSpec((1, H2), lambda i: (0, 0)),
            pl.BlockSpec((H2, D), lambda i: (0, 0)),
            pl.BlockSpec(memory_space=pltpu.SMEM),
        ],
        out_specs=pl.BlockSpec((BB,), lambda i: (i,)),
        out_shape=jax.ShapeDtypeStruct((B,), jnp.float32),
    )(x, lin_sum, W1, b1, W2, b2, W3p, scal)


def kernel(features, emb_fm, lin_fm, bias, emb_dnn, W1, b1, W2, b2, W3, b3,
           Wf, bf):
    feats = features.astype(jnp.int32)
    lin_idx = feats.T + (jnp.arange(F, dtype=jnp.int32) * V)[:, None]
    idx = lin_idx.reshape(F * B)

    packed = _pack(emb_dnn.reshape(F * V, D), emb_fm.reshape(F * V, D))
    rows, lin_sum = _sc_gather(packed, lin_fm.reshape(F * V), idx, lin_idx)

    x = rows.reshape(F, B, D)

    W3p = jnp.pad(W3, ((0, 0), (0, D - 1)))
    scal = jnp.concatenate([bias, b3, Wf[0], Wf[1], bf])
    out = _tc_forward(x, lin_sum, W1.astype(jnp.bfloat16),
                      b1.reshape(1, H1), W2.astype(jnp.bfloat16),
                      b2.reshape(1, H2), W3p.astype(jnp.bfloat16), scal)
    return out


# speedup vs baseline: 103.1933x; 1.0089x over previous
import functools

import jax
import jax.numpy as jnp
from jax import lax
from jax.experimental import pallas as pl
from jax.experimental.pallas import tpu as pltpu
from jax.experimental.pallas import tpu_sc as plsc

B = 4096
F = 26
V = 1000
D = 128
DNN_IN = F * D
H1, H2 = 1024, 512
HD = D // 2

NC = 2
NS = 16
NW = NC * NS
BPW = B // NW
IPW = BPW * F
CH = 4
CHI = CH * F
NCHUNK = BPW // CH

RB = 2600
BB = 512
NBLK = B // BB

def _pack_body(dnn_ref, fm_ref, out_ref):
    def words(x):
        xb = x.astype(jnp.bfloat16).astype(jnp.float32)
        u = lax.bitcast_convert_type(xb, jnp.uint32)
        return (u[:, :HD] >> 16) | u[:, HD:]
    w = jnp.concatenate([words(dnn_ref[...]), words(fm_ref[...])], axis=1)
    out_ref[...] = lax.bitcast_convert_type(w, jnp.int32)


def _pack(dnn_tab, fm_tab):
    return pl.pallas_call(
        _pack_body,
        grid=(F * V // RB,),
        in_specs=[
            pl.BlockSpec((RB, D), lambda i: (i, 0)),
            pl.BlockSpec((RB, D), lambda i: (i, 0)),
        ],
        out_specs=pl.BlockSpec((RB, D), lambda i: (i, 0)),
        out_shape=jax.ShapeDtypeStruct((F * V, D), jnp.int32),
    )(dnn_tab, fm_tab)


def _sc_gather(packed_tab, lin_tab, idx, lin_idx):
    mesh = plsc.VectorSubcoreMesh(core_axis_name="c", subcore_axis_name="s",
                                  num_cores=NC, num_subcores=NS)

    @functools.partial(
        pl.kernel,
        mesh=mesh,
        compiler_params=pltpu.CompilerParams(needs_layout_passes=False),
        out_type=(
            jax.ShapeDtypeStruct((F * B, D), jnp.int32),
            jax.ShapeDtypeStruct((B,), jnp.float32),
        ),
        scratch_types=[
            pltpu.VMEM((IPW,), jnp.int32),
            pltpu.VMEM((CHI, D), jnp.int32),
            pltpu.VMEM((CHI, D), jnp.int32),
            pltpu.VMEM((F * V,), jnp.float32),
            pltpu.VMEM((F, BPW), jnp.int32),
            pltpu.VMEM((BPW,), jnp.float32),
            pltpu.SemaphoreType.DMA,
            pltpu.SemaphoreType.DMA,
        ],
    )
    def k(tab_hbm, lin_hbm, idx_hbm, lin_idx_hbm,
          rows_out, lin_out,
          idx_v, b0, b1, lin_tab_v, lin_idx_v, lin_sum_v, s0, s1):
        wid = lax.axis_index("s") * NC + lax.axis_index("c")
        base = wid * IPW

        pltpu.sync_copy(idx_hbm.at[pl.ds(base, IPW)], idx_v)
        pltpu.sync_copy(lin_hbm, lin_tab_v)
        pltpu.sync_copy(lin_idx_hbm.at[:, pl.ds(wid * BPW, BPW)], lin_idx_v)

        def gather(j, buf, sem):
            pltpu.async_copy(tab_hbm.at[idx_v.at[pl.ds(j * CHI, CHI)]],
                             buf, sem)

        def wait(j, buf, sem):
            pltpu.make_async_copy(tab_hbm.at[idx_v.at[pl.ds(j * CHI, CHI)]],
                                  buf, sem).wait()

        def writeback(j, buf):
            pltpu.sync_copy(buf, rows_out.at[pl.ds(base + j * CHI, CHI)])

        gather(0, b0, s0)
        for g in range(BPW // 16):
            acc = jnp.zeros((16,), jnp.float32)
            for f in range(F):
                iv = lin_idx_v[f, pl.ds(g * 16, 16)]
                acc = acc + plsc.load_gather(lin_tab_v, [iv])
            lin_sum_v[pl.ds(g * 16, 16)] = acc
        pltpu.sync_copy(lin_sum_v, lin_out.at[pl.ds(wid * BPW, BPW)])

        def pair(p, _):
            j0 = 2 * p
            wait(j0, b0, s0)
            gather(j0 + 1, b1, s1)
            writeback(j0, b0)
            wait(j0 + 1, b1, s1)
            gather(jnp.minimum(j0 + 2, NCHUNK - 1), b0, s0)
            writeback(j0 + 1, b1)
            return 0

        lax.fori_loop(0, NCHUNK // 2, pair, 0)
        wait(NCHUNK - 1, b0, s0)

    return k(packed_tab, lin_tab, idx, lin_idx)


def _tc_body(x_ref, lin_ref, w1_ref, b1_ref, w2_ref, b2_ref,
             w3_ref, scal_ref, out_ref):
    def unpack(w):
        lo = lax.bitcast_convert_type(w << 16, jnp.float32)
        hi = lax.bitcast_convert_type(w & jnp.uint32(0xFFFF0000),
                                      jnp.float32)
        return jnp.concatenate([lo, hi], axis=1)

    s = jnp.zeros((BB, D), jnp.float32)
    q = jnp.zeros((BB, D), jnp.float32)
    dnn_parts = []
    for f in range(F):
        w = lax.bitcast_convert_type(x_ref[f], jnp.uint32)
        dnn_parts.append(unpack(w[:, :HD]).astype(jnp.bfloat16))
        e = unpack(w[:, HD:])
        s = s + e
        q = q + e * e
    fm_term = 0.5 * jnp.sum(s * s - q, axis=1)

    xd = jnp.concatenate(dnn_parts, axis=1)
    h = jnp.dot(xd, w1_ref[...], preferred_element_type=jnp.float32)
    h = jnp.maximum(h + b1_ref[...], 0.0)
    h = jnp.dot(h.astype(jnp.bfloat16), w2_ref[...],
                preferred_element_type=jnp.float32)
    h = jnp.maximum(h + b2_ref[...], 0.0)
    dnn_mat = jnp.dot(h.astype(jnp.bfloat16), w3_ref[...],
                      preferred_element_type=jnp.float32)
    dnn = jnp.sum(dnn_mat, axis=1)

    bias0 = scal_ref[0]
    b3 = scal_ref[1]
    wf0 = scal_ref[2]
    wf1 = scal_ref[3]
    bf = scal_ref[4]
    fm_output = bias0 + lin_ref[...] + fm_term
    logit = wf0 * fm_output + wf1 * (dnn + b3) + bf
    out_ref[...] = 1.0 / (1.0 + jnp.exp(-logit))


def _tc_forward(x, lin_sum, W1, b1, W2, b2, W3p, scal):
    return pl.pallas_call(
        _tc_body,
        grid=(NBLK,),
        in_specs=[
            pl.BlockSpec((F, BB, D), lambda i: (0, i, 0)),
            pl.BlockSpec((BB,), lambda i: (i,)),
            pl.BlockSpec((DNN_IN, H1), lambda i: (0, 0)),
            pl.BlockSpec((1, H1), lambda i: (0, 0)),
            pl.BlockSpec((H1, H2), lambda i: (0, 0)),
            pl.BlockSpec((1, H2), lambda i: (0, 0)),
            pl.BlockSpec((H2, D), lambda i: (0, 0)),
            pl.BlockSpec(memory_space=pltpu.SMEM),
        ],
        out_specs=pl.BlockSpec((BB,), lambda i: (i,)),
        out_shape=jax.ShapeDtypeStruct((B,), jnp.float32),
    )(x, lin_sum, W1, b1, W2, b2, W3p, scal)


def kernel(features, emb_fm, lin_fm, bias, emb_dnn, W1, b1, W2, b2, W3, b3,
           Wf, bf):
    feats = features.astype(jnp.int32)
    lin_idx = feats.T + (jnp.arange(F, dtype=jnp.int32) * V)[:, None]
    idx = lin_idx.reshape(F * B)

    packed = _pack(emb_dnn.reshape(F * V, D), emb_fm.reshape(F * V, D))
    rows, lin_sum = _sc_gather(packed, lin_fm.reshape(F * V), idx, lin_idx)

    x = rows.reshape(F, B, D)

    W3p = jnp.pad(W3, ((0, 0), (0, D - 1)))
    scal = jnp.concatenate([bias, b3, Wf[0], Wf[1], bf])
    out = _tc_forward(x, lin_sum, W1.astype(jnp.bfloat16),
                      b1.reshape(1, H1), W2.astype(jnp.bfloat16),
                      b2.reshape(1, H2), W3p.astype(jnp.bfloat16), scal)
    return out
